# Initial kernel scaffold; baseline (speedup 1.0000x reference)
import jax
import jax.numpy as jnp
from jax.experimental import pallas as pl


def kernel(node_feature, edge_index, edge_feature, batch, params):
    raise NotImplementedError("write your pallas kernel here")


# speedup vs baseline: 2.2815x; 2.2815x over previous
import functools

import jax
import jax.numpy as jnp
from jax import lax
from jax.experimental import pallas as pl
from jax.experimental.pallas import tpu as pltpu
from jax.experimental.pallas import tpu_sc as plsc

_N = 10000
_NP = 10240
_E = 160000
_EP = 163840
_D = 128
_ED = 16
_H = 8
_G = 16
_NC = 2
_NS = 16
_NW = _NC * _NS
_RN = 80
_NR = 128
_RPW = _NR // _NW
_AR = _RN + 8

_mesh = plsc.VectorSubcoreMesh(core_axis_name="c", subcore_axis_name="s")


@functools.lru_cache(maxsize=None)
def _gather_fn(V, width, chunk):
  per_w = _EP // _NW
  n_ch = per_w // chunk

  def body(table_hbm, idx_hbm, out_hbm, idx_v, rows_v, sem):
    wid = lax.axis_index("s") * _NC + lax.axis_index("c")
    base = wid * per_w

    def step(ci, c):
      e0 = base + ci * chunk
      pltpu.sync_copy(idx_hbm.at[pl.ds(e0, chunk)], idx_v)
      pltpu.async_copy(table_hbm.at[idx_v], rows_v, sem).wait()
      pltpu.sync_copy(rows_v, out_hbm.at[pl.ds(e0, chunk)])
      return c

    lax.fori_loop(0, n_ch, step, 0)

  return pl.kernel(
      body,
      out_type=jax.ShapeDtypeStruct((_EP, width), jnp.float32),
      mesh=_mesh,
      scratch_types=[
          pltpu.VMEM((chunk,), jnp.int32),
          pltpu.VMEM((chunk, width), jnp.float32),
          pltpu.SemaphoreType.DMA,
      ])


def _sc_gather(table, idx, chunk):
  return _gather_fn(table.shape[0], table.shape[1], chunk)(table, idx)


@functools.lru_cache(maxsize=None)
def _reduce_fn(width, chunk, do_max, do_cnt):
  def body(*refs):
    it = iter(refs)
    msg_hbm = next(it); dst_hbm = next(it); off_hbm = next(it)
    zs_hbm = next(it)
    zm_hbm = next(it) if do_max else None
    zc_hbm = next(it) if do_cnt else None
    sum_hbm = next(it)
    max_hbm = next(it) if do_max else None
    cnt_hbm = next(it) if do_cnt else None
    offs_v = next(it); dstv_v = next(it); lrow_v = next(it)
    msg_v = next(it)
    acc_s = next(it)
    acc_m = next(it) if do_max else None
    acc_c = next(it) if do_cnt else None
    sem = next(it)

    wid = lax.axis_index("s") * _NC + lax.axis_index("c")

    def rbody(r, cr):
      ridx = wid * _RPW + r
      base = ridx * _RN
      pltpu.sync_copy(off_hbm.at[ridx], offs_v)
      pltpu.sync_copy(zs_hbm, acc_s)
      if do_max:
        pltpu.sync_copy(zm_hbm, acc_m)
      if do_cnt:
        pltpu.sync_copy(zc_hbm, acc_c)
      orow = offs_v[pl.ds(0, 16)]
      e0 = orow[0]
      e1 = orow[1]
      e0a = (e0 // chunk) * chunk
      nch = (e1 - e0a + chunk - 1) // chunk

      def step(ci, c, e0a=e0a, base=base):
        ea = e0a + ci * chunk
        pltpu.sync_copy(dst_hbm.at[pl.ds(ea, chunk)], dstv_v)
        pltpu.async_copy(msg_hbm.at[pl.ds(ea, chunk)], msg_v, sem).wait()
        for jj in range(chunk // 16):
          dv = dstv_v[pl.ds(jj * 16, 16)]
          lr = dv - base
          ok = jnp.logical_and(lr >= 0, lr < _RN)
          lrow_v[pl.ds(jj * 16, 16)] = jnp.where(ok, lr, _RN)

        def inner(i2, c2):
          rvec = lrow_v[pl.ds(i2 * 16, 16)]
          for kk in range(16):
            row = rvec[kk]
            for j in range(width // 16):
              sl = pl.ds(j * 16, 16)
              mv = msg_v[i2 * 16 + kk, sl]
              acc_s[row, sl] += mv
              if do_max:
                acc_m[row, sl] = jnp.maximum(acc_m[row, sl], mv)
            if do_cnt:
              c16 = pl.ds(0, 16)
              acc_c[row, c16] += 1.0
          return c2

        lax.fori_loop(0, chunk // 16, inner, 0)
        return c

      lax.fori_loop(0, nch, step, 0)
      pltpu.sync_copy(acc_s.at[pl.ds(0, _RN)], sum_hbm.at[pl.ds(base, _RN)])
      if do_max:
        pltpu.sync_copy(acc_m.at[pl.ds(0, _RN)], max_hbm.at[pl.ds(base, _RN)])
      if do_cnt:
        pltpu.sync_copy(acc_c.at[pl.ds(0, _RN)], cnt_hbm.at[pl.ds(base, _RN)])
      return cr

    lax.fori_loop(0, _RPW, rbody, 0)

  outs = [jax.ShapeDtypeStruct((_NP, width), jnp.float32)]
  if do_max:
    outs.append(jax.ShapeDtypeStruct((_NP, width), jnp.float32))
  if do_cnt:
    outs.append(jax.ShapeDtypeStruct((_NP, 16), jnp.float32))
  scratch = [
      pltpu.VMEM((16,), jnp.int32),
      pltpu.VMEM((chunk,), jnp.int32),
      pltpu.VMEM((chunk,), jnp.int32),
      pltpu.VMEM((chunk, width), jnp.float32),
      pltpu.VMEM((_AR, width), jnp.float32),
  ]
  if do_max:
    scratch.append(pltpu.VMEM((_AR, width), jnp.float32))
  if do_cnt:
    scratch.append(pltpu.VMEM((_AR, 16), jnp.float32))
  scratch.append(pltpu.SemaphoreType.DMA)

  return pl.kernel(body, out_type=tuple(outs), mesh=_mesh,
                   scratch_types=scratch)


def _sc_reduce(msg, dst_s, off, chunk, do_max, do_cnt):
  width = msg.shape[1]
  fn = _reduce_fn(width, chunk, do_max, do_cnt)
  args = [msg, dst_s, off, jnp.zeros((_AR, width), jnp.float32)]
  if do_max:
    args.append(jnp.full((_AR, width), -1e38, jnp.float32))
  if do_cnt:
    args.append(jnp.zeros((_AR, 16), jnp.float32))
  return fn(*args)


def _mm(x, W, b, bm):
  M, K = x.shape
  F = W.shape[1]

  def body(x_ref, w_ref, b_ref, o_ref):
    o_ref[...] = jnp.dot(x_ref[...], w_ref[...],
                         preferred_element_type=jnp.float32) + b_ref[...]

  return pl.pallas_call(
      body,
      grid=(M // bm,),
      in_specs=[pl.BlockSpec((bm, K), lambda i: (i, 0)),
                pl.BlockSpec((K, F), lambda i: (0, 0)),
                pl.BlockSpec((1, F), lambda i: (0, 0))],
      out_specs=pl.BlockSpec((bm, F), lambda i: (i, 0)),
      out_shape=jax.ShapeDtypeStruct((M, F), jnp.float32),
  )(x, W, b.reshape(1, F))


def _msg_tc(kd, qs, ep, vs, be=2048):
  def body(a, b, c, d, o):
    o[...] = jax.nn.sigmoid(a[...] + b[...] + c[...]) * d[...]

  spec = pl.BlockSpec((be, _D), lambda i: (i, 0))
  return pl.pallas_call(
      body, grid=(_EP // be,),
      in_specs=[spec] * 4, out_specs=spec,
      out_shape=jax.ShapeDtypeStruct((_EP, _D), jnp.float32))(kd, qs, ep, vs)


def _combine_tc(sm, mx, ct, xs, do_stats, bm=1000):
  nb = _N // bm
  F = 2 * _D

  def body(s_ref, m_ref, c_ref, xs_ref, o_ref, st_ref, acc):
    i = pl.program_id(0)
    deg = jnp.maximum(c_ref[...][:, :1], 1.0)
    mean = s_ref[...] / deg
    mraw = m_ref[...]
    mv = jnp.where(mraw < -9e37, 0.0, mraw)
    x1 = jnp.concatenate([mean, mv], axis=1) + xs_ref[...]
    if do_stats:
      x1 = x1 * jnp.tanh(jax.nn.softplus(x1))

      @pl.when(i == 0)
      def _():
        acc[...] = jnp.zeros_like(acc)

      acc[0:1, :] += jnp.sum(x1, axis=0, keepdims=True)
      acc[1:2, :] += jnp.sum(x1 * x1, axis=0, keepdims=True)

      @pl.when(i == nb - 1)
      def _():
        st_ref[...] = acc[...]
    else:
      @pl.when(i == 0)
      def _():
        st_ref[...] = jnp.zeros_like(st_ref)
    o_ref[...] = x1

  return pl.pallas_call(
      body, grid=(nb,),
      in_specs=[pl.BlockSpec((bm, _D), lambda i: (i, 0)),
                pl.BlockSpec((bm, _D), lambda i: (i, 0)),
                pl.BlockSpec((bm, 16), lambda i: (i, 0)),
                pl.BlockSpec((bm, F), lambda i: (i, 0))],
      out_specs=[pl.BlockSpec((bm, F), lambda i: (i, 0)),
                 pl.BlockSpec((8, F), lambda i: (0, 0))],
      out_shape=[jax.ShapeDtypeStruct((_N, F), jnp.float32),
                 jax.ShapeDtypeStruct((8, F), jnp.float32)],
      scratch_shapes=[pltpu.VMEM((8, F), jnp.float32)],
  )(sm, mx, ct, xs)


def _norm_apply_tc(x1, stats, weight, bias, mscale, bm=1000):
  F = 2 * _D

  def body(x_ref, st_ref, w_ref, b_ref, ms_ref, o_ref):
    mean = st_ref[0:1, :] / _N
    ex2 = st_ref[1:2, :] / _N
    ctr = ms_ref[...] * mean
    var = ex2 - 2.0 * ctr * mean + ctr * ctr
    scale = w_ref[...] * jax.lax.rsqrt(var + 1e-5)
    shift = b_ref[...] - ctr * scale
    o_ref[...] = x_ref[...] * scale + shift

  return pl.pallas_call(
      body, grid=(_N // bm,),
      in_specs=[pl.BlockSpec((bm, F), lambda i: (i, 0)),
                pl.BlockSpec((8, F), lambda i: (0, 0)),
                pl.BlockSpec((1, F), lambda i: (0, 0)),
                pl.BlockSpec((1, F), lambda i: (0, 0)),
                pl.BlockSpec((1, F), lambda i: (0, 0))],
      out_specs=pl.BlockSpec((bm, F), lambda i: (i, 0)),
      out_shape=jax.ShapeDtypeStruct((_N, F), jnp.float32),
  )(x1, stats, weight.reshape(1, F), bias.reshape(1, F),
    mscale.reshape(1, F))


def _alpha_tc(xls, xrd, eg, att, be=1024):
  W = _H * _D

  def body(a_ref, b_ref, c_ref, att_ref, o_ref):
    m = a_ref[...] + b_ref[...] + c_ref[...]
    m = jnp.where(m >= 0, m, 0.2 * m)
    m3 = m.reshape(be, _H, _D) * att_ref[...][None]
    al = jnp.sum(m3, axis=2)
    o_ref[...] = jnp.concatenate(
        [al, jnp.zeros((be, 128 - _H), jnp.float32)], axis=1)

  spec = pl.BlockSpec((be, W), lambda i: (i, 0))
  return pl.pallas_call(
      body, grid=(_EP // be,),
      in_specs=[spec, spec, spec, pl.BlockSpec((_H, _D), lambda i: (0, 0))],
      out_specs=pl.BlockSpec((be, 128), lambda i: (i, 0)),
      out_shape=jax.ShapeDtypeStruct((_EP, 128), jnp.float32),
  )(xls, xrd, eg, att)


def _aexp_tc(alpha, amax_d, be=2048):
  def body(a_ref, m_ref, o_ref):
    o_ref[...] = jnp.exp(a_ref[...] - m_ref[...])

  spec = pl.BlockSpec((be, 128), lambda i: (i, 0))
  return pl.pallas_call(
      body, grid=(_EP // be,), in_specs=[spec, spec], out_specs=spec,
      out_shape=jax.ShapeDtypeStruct((_EP, 128), jnp.float32))(alpha, amax_d)


def _wdiv_tc(aexp, den_d, be=2048):
  def body(a_ref, d_ref, o_ref):
    o_ref[...] = a_ref[...] / (d_ref[...] + 1e-16)

  spec = pl.BlockSpec((be, 128), lambda i: (i, 0))
  return pl.pallas_call(
      body, grid=(_EP // be,), in_specs=[spec, spec], out_specs=spec,
      out_shape=jax.ShapeDtypeStruct((_EP, 128), jnp.float32))(aexp, den_d)


def _scale_tc(xls, wgt, be=1024):
  W = _H * _D

  def body(x_ref, w_ref, o_ref):
    w8 = w_ref[...][:, :_H]
    o_ref[...] = (x_ref[...].reshape(be, _H, _D)
                  * w8[:, :, None]).reshape(be, W)

  return pl.pallas_call(
      body, grid=(_EP // be,),
      in_specs=[pl.BlockSpec((be, W), lambda i: (i, 0)),
                pl.BlockSpec((be, 128), lambda i: (i, 0))],
      out_specs=pl.BlockSpec((be, W), lambda i: (i, 0)),
      out_shape=jax.ShapeDtypeStruct((_EP, W), jnp.float32))(xls, wgt)


def _pool_tc(g, batch2, Wg, bg, bm=1000):
  W = _H * _D
  nb = _N // bm

  def body(g_ref, b_ref, wg_ref, bg_ref, o_ref, acc, cacc):
    i = pl.program_id(0)

    @pl.when(i == 0)
    def _():
      acc[...] = jnp.zeros_like(acc)
      cacc[...] = jnp.zeros_like(cacc)

    oh = (b_ref[...] == lax.broadcasted_iota(jnp.int32, (1, _G), 1)
          ).astype(jnp.float32)
    acc[...] += lax.dot_general(oh, g_ref[...], (((0,), (0,)), ((), ())),
                                preferred_element_type=jnp.float32)
    cacc[0:1, :] += jnp.sum(oh, axis=0, keepdims=True)

    @pl.when(i == nb - 1)
    def _():
      cnt = jnp.maximum(cacc[0:1, :], 1.0).reshape(_G, 1)
      gm = acc[...] / cnt
      o_ref[...] = jnp.dot(gm, wg_ref[...],
                           preferred_element_type=jnp.float32) + bg_ref[...]

  return pl.pallas_call(
      body, grid=(nb,),
      in_specs=[pl.BlockSpec((bm, W), lambda i: (i, 0)),
                pl.BlockSpec((bm, 1), lambda i: (i, 0)),
                pl.BlockSpec((W, _D), lambda i: (0, 0)),
                pl.BlockSpec((1, _D), lambda i: (0, 0))],
      out_specs=pl.BlockSpec((_G, _D), lambda i: (0, 0)),
      out_shape=jax.ShapeDtypeStruct((_G, _D), jnp.float32),
      scratch_shapes=[pltpu.VMEM((_G, W), jnp.float32),
                      pltpu.VMEM((8, _G), jnp.float32)],
  )(g, batch2, Wg, bg.reshape(1, _D))


def kernel(node_feature, edge_index, edge_feature, batch, params):
  src = edge_index[0].astype(jnp.int32)
  dst = edge_index[1].astype(jnp.int32)
  perm = jnp.argsort(dst)
  dst_r = dst[perm]
  src_r = src[perm]
  ea_r = edge_feature[perm]
  pad = _EP - _E
  dst_s = jnp.concatenate([dst_r, jnp.full((pad,), _NP - 1, jnp.int32)])
  dst_g = jnp.concatenate([dst_r, jnp.zeros((pad,), jnp.int32)])
  src_s = jnp.concatenate([src_r, jnp.zeros((pad,), jnp.int32)])
  ea_s = jnp.concatenate([ea_r, jnp.zeros((pad, _ED), jnp.float32)])
  bounds = (jnp.arange(_NR + 1, dtype=jnp.int32) * _RN).astype(jnp.int32)
  off = jnp.searchsorted(dst_s, bounds).astype(jnp.int32)
  gidx = jnp.clip(jnp.arange(_NR)[:, None] + jnp.arange(16)[None, :],
                  0, _NR)
  off = off[gidx]

  x = node_feature
  for l in range(3):
    p = params['convs'][l]
    k = _mm(x, p['Wk'], p['bk'], 1000)
    q = _mm(x, p['Wq'], p['bq'], 1000)
    v = _mm(x, p['Wv'], p['bv'], 1000)
    xs = _mm(x, p['Ws'], p['bs'], 1000)
    ep = _mm(ea_s, p['We'], p['be'], 2048)
    kd = _sc_gather(k, dst_g, 128)
    qs = _sc_gather(q, src_s, 128)
    vs = _sc_gather(v, src_s, 128)
    msg = _msg_tc(kd, qs, ep, vs)
    sm, mx, ct = _sc_reduce(msg, dst_s, off, 128, True, True)
    x1, stats = _combine_tc(sm[:_N], mx[:_N], ct[:_N], xs, l < 2)
    if l < 2:
      np_ = params['norms'][l]
      x = _norm_apply_tc(x1, stats, np_['weight'], np_['bias'],
                         np_['mean_scale'])
    else:
      x = x1

  nl = params['node_linear']
  local = _mm(x, nl['W'], nl['b'], 1000)

  gp = params['gat']
  xl = _mm(local, gp['Wl'], gp['bl'], 1000)
  xr = _mm(local, gp['Wr'], gp['br'], 1000)
  eg = _mm(ea_s, gp['We'], gp['be'], 2048)
  xls = _sc_gather(xl, src_s, 64)
  xrd = _sc_gather(xr, dst_g, 64)
  alpha = _alpha_tc(xls, xrd, eg, gp['att'])
  (amax,) = _sc_reduce(alpha, dst_s, off, 128, True, False)[1:]
  amax_d = _sc_gather(amax, dst_s, 128)
  aexp = _aexp_tc(alpha, amax_d)
  (den,) = _sc_reduce(aexp, dst_s, off, 128, False, False)
  den_d = _sc_gather(den, dst_s, 128)
  wgt = _wdiv_tc(aexp, den_d)
  smsg = _scale_tc(xls, wgt)
  (g,) = _sc_reduce(smsg, dst_s, off, 32, False, False)

  gl = params['graph_linear']
  glob = _pool_tc(g[:_N], batch.reshape(_N, 1).astype(jnp.int32),
                  gl['W'], gl['b'], 1000)
  return local, glob


# speedup vs baseline: 2.4564x; 1.0767x over previous
import functools

import jax
import jax.numpy as jnp
from jax import lax
from jax.experimental import pallas as pl
from jax.experimental.pallas import tpu as pltpu
from jax.experimental.pallas import tpu_sc as plsc

_N = 10000
_NP = 10240
_E = 160000
_EP = 163840
_D = 128
_ED = 16
_H = 8
_G = 16
_NC = 2
_NS = 16
_NW = _NC * _NS
_RN = 80
_NR = 128
_RPW = _NR // _NW
_AR = _RN + 8

_mesh = plsc.VectorSubcoreMesh(core_axis_name="c", subcore_axis_name="s")


@functools.lru_cache(maxsize=None)
def _gather_fn(V, width, chunk):
  per_w = _EP // _NW
  n_ch = per_w // chunk

  n2 = n_ch // 2

  def body(table_hbm, idx_hbm, out_hbm, idx_v0, idx_v1, rows_v0, rows_v1,
           sem0, sem1):
    wid = lax.axis_index("s") * _NC + lax.axis_index("c")
    base = wid * per_w

    pltpu.sync_copy(idx_hbm.at[pl.ds(base, chunk)], idx_v0)
    pltpu.async_copy(table_hbm.at[idx_v0], rows_v0, sem0)

    def step(ci, c):
      ea = base + (2 * ci) * chunk
      eb = ea + chunk
      pltpu.sync_copy(idx_hbm.at[pl.ds(eb, chunk)], idx_v1)
      pltpu.async_copy(table_hbm.at[idx_v1], rows_v1, sem1)
      pltpu.make_async_copy(table_hbm.at[idx_v0], rows_v0, sem0).wait()
      pltpu.sync_copy(rows_v0, out_hbm.at[pl.ds(ea, chunk)])

      @pl.when(ci < n2 - 1)
      def _():
        ea2 = ea + 2 * chunk
        pltpu.sync_copy(idx_hbm.at[pl.ds(ea2, chunk)], idx_v0)
        pltpu.async_copy(table_hbm.at[idx_v0], rows_v0, sem0)

      pltpu.make_async_copy(table_hbm.at[idx_v1], rows_v1, sem1).wait()
      pltpu.sync_copy(rows_v1, out_hbm.at[pl.ds(eb, chunk)])
      return c

    lax.fori_loop(0, n2, step, 0)

  return pl.kernel(
      body,
      out_type=jax.ShapeDtypeStruct((_EP, width), jnp.float32),
      mesh=_mesh,
      scratch_types=[
          pltpu.VMEM((chunk,), jnp.int32),
          pltpu.VMEM((chunk,), jnp.int32),
          pltpu.VMEM((chunk, width), jnp.float32),
          pltpu.VMEM((chunk, width), jnp.float32),
          pltpu.SemaphoreType.DMA,
          pltpu.SemaphoreType.DMA,
      ])


def _sc_gather(table, idx, chunk):
  return _gather_fn(table.shape[0], table.shape[1], chunk)(table, idx)


@functools.lru_cache(maxsize=None)
def _reduce_fn(width, chunk, do_max, do_cnt):
  def body(*refs):
    it = iter(refs)
    msg_hbm = next(it); dst_hbm = next(it); off_hbm = next(it)
    zs_hbm = next(it)
    zm_hbm = next(it) if do_max else None
    zc_hbm = next(it) if do_cnt else None
    sum_hbm = next(it)
    max_hbm = next(it) if do_max else None
    cnt_hbm = next(it) if do_cnt else None
    offs_v = next(it); dstv_v = next(it); lrow_v = next(it)
    msg_v = next(it)
    acc_s = next(it)
    acc_m = next(it) if do_max else None
    acc_c = next(it) if do_cnt else None
    sem = next(it)

    wid = lax.axis_index("s") * _NC + lax.axis_index("c")

    def rbody(r, cr):
      ridx = wid * _RPW + r
      base = ridx * _RN
      pltpu.sync_copy(off_hbm.at[ridx], offs_v)
      pltpu.sync_copy(zs_hbm, acc_s)
      if do_max:
        pltpu.sync_copy(zm_hbm, acc_m)
      if do_cnt:
        pltpu.sync_copy(zc_hbm, acc_c)
      orow = offs_v[pl.ds(0, 16)]
      e0 = orow[0]
      e1 = orow[1]
      e0a = (e0 // chunk) * chunk
      nch = (e1 - e0a + chunk - 1) // chunk

      def step(ci, c, e0a=e0a, base=base):
        ea = e0a + ci * chunk
        pltpu.sync_copy(dst_hbm.at[pl.ds(ea, chunk)], dstv_v)
        pltpu.async_copy(msg_hbm.at[pl.ds(ea, chunk)], msg_v, sem).wait()
        for jj in range(chunk // 16):
          dv = dstv_v[pl.ds(jj * 16, 16)]
          lr = dv - base
          ok = jnp.logical_and(lr >= 0, lr < _RN)
          lrow_v[pl.ds(jj * 16, 16)] = jnp.where(ok, lr, _RN)

        def inner(i2, c2):
          rvec = lrow_v[pl.ds(i2 * 16, 16)]
          for kk in range(16):
            row = rvec[kk]
            for j in range(width // 16):
              sl = pl.ds(j * 16, 16)
              mv = msg_v[i2 * 16 + kk, sl]
              acc_s[row, sl] += mv
              if do_max:
                acc_m[row, sl] = jnp.maximum(acc_m[row, sl], mv)
            if do_cnt:
              c16 = pl.ds(0, 16)
              acc_c[row, c16] += 1.0
          return c2

        lax.fori_loop(0, chunk // 16, inner, 0)
        return c

      lax.fori_loop(0, nch, step, 0)
      pltpu.sync_copy(acc_s.at[pl.ds(0, _RN)], sum_hbm.at[pl.ds(base, _RN)])
      if do_max:
        pltpu.sync_copy(acc_m.at[pl.ds(0, _RN)], max_hbm.at[pl.ds(base, _RN)])
      if do_cnt:
        pltpu.sync_copy(acc_c.at[pl.ds(0, _RN)], cnt_hbm.at[pl.ds(base, _RN)])
      return cr

    lax.fori_loop(0, _RPW, rbody, 0)

  outs = [jax.ShapeDtypeStruct((_NP, width), jnp.float32)]
  if do_max:
    outs.append(jax.ShapeDtypeStruct((_NP, width), jnp.float32))
  if do_cnt:
    outs.append(jax.ShapeDtypeStruct((_NP, 16), jnp.float32))
  scratch = [
      pltpu.VMEM((16,), jnp.int32),
      pltpu.VMEM((chunk,), jnp.int32),
      pltpu.VMEM((chunk,), jnp.int32),
      pltpu.VMEM((chunk, width), jnp.float32),
      pltpu.VMEM((_AR, width), jnp.float32),
  ]
  if do_max:
    scratch.append(pltpu.VMEM((_AR, width), jnp.float32))
  if do_cnt:
    scratch.append(pltpu.VMEM((_AR, 16), jnp.float32))
  scratch.append(pltpu.SemaphoreType.DMA)

  return pl.kernel(body, out_type=tuple(outs), mesh=_mesh,
                   scratch_types=scratch)


def _sc_reduce(msg, dst_s, off, chunk, do_max, do_cnt):
  width = msg.shape[1]
  fn = _reduce_fn(width, chunk, do_max, do_cnt)
  args = [msg, dst_s, off, jnp.zeros((_AR, width), jnp.float32)]
  if do_max:
    args.append(jnp.full((_AR, width), -1e38, jnp.float32))
  if do_cnt:
    args.append(jnp.zeros((_AR, 16), jnp.float32))
  return fn(*args)


def _mm(x, W, b, bm):
  M, K = x.shape
  F = W.shape[1]

  def body(x_ref, w_ref, b_ref, o_ref):
    o_ref[...] = jnp.dot(x_ref[...], w_ref[...],
                         preferred_element_type=jnp.float32) + b_ref[...]

  return pl.pallas_call(
      body,
      grid=(M // bm,),
      in_specs=[pl.BlockSpec((bm, K), lambda i: (i, 0)),
                pl.BlockSpec((K, F), lambda i: (0, 0)),
                pl.BlockSpec((1, F), lambda i: (0, 0))],
      out_specs=pl.BlockSpec((bm, F), lambda i: (i, 0)),
      out_shape=jax.ShapeDtypeStruct((M, F), jnp.float32),
  )(x, W, b.reshape(1, F))


def _msg_tc(kd, qs, ep, vs, be=2048):
  def body(a, b, c, d, o):
    o[...] = jax.nn.sigmoid(a[...] + b[...] + c[...]) * d[...]

  spec = pl.BlockSpec((be, _D), lambda i: (i, 0))
  return pl.pallas_call(
      body, grid=(_EP // be,),
      in_specs=[spec] * 4, out_specs=spec,
      out_shape=jax.ShapeDtypeStruct((_EP, _D), jnp.float32))(kd, qs, ep, vs)


def _combine_tc(sm, mx, ct, xs, do_stats, bm=1000):
  nb = _N // bm
  F = 2 * _D

  def body(s_ref, m_ref, c_ref, xs_ref, o_ref, st_ref, acc):
    i = pl.program_id(0)
    deg = jnp.maximum(c_ref[...][:, :1], 1.0)
    mean = s_ref[...] / deg
    mraw = m_ref[...]
    mv = jnp.where(mraw < -9e37, 0.0, mraw)
    x1 = jnp.concatenate([mean, mv], axis=1) + xs_ref[...]
    if do_stats:
      x1 = x1 * jnp.tanh(jax.nn.softplus(x1))

      @pl.when(i == 0)
      def _():
        acc[...] = jnp.zeros_like(acc)

      acc[0:1, :] += jnp.sum(x1, axis=0, keepdims=True)
      acc[1:2, :] += jnp.sum(x1 * x1, axis=0, keepdims=True)

      @pl.when(i == nb - 1)
      def _():
        st_ref[...] = acc[...]
    else:
      @pl.when(i == 0)
      def _():
        st_ref[...] = jnp.zeros_like(st_ref)
    o_ref[...] = x1

  return pl.pallas_call(
      body, grid=(nb,),
      in_specs=[pl.BlockSpec((bm, _D), lambda i: (i, 0)),
                pl.BlockSpec((bm, _D), lambda i: (i, 0)),
                pl.BlockSpec((bm, 16), lambda i: (i, 0)),
                pl.BlockSpec((bm, F), lambda i: (i, 0))],
      out_specs=[pl.BlockSpec((bm, F), lambda i: (i, 0)),
                 pl.BlockSpec((8, F), lambda i: (0, 0))],
      out_shape=[jax.ShapeDtypeStruct((_N, F), jnp.float32),
                 jax.ShapeDtypeStruct((8, F), jnp.float32)],
      scratch_shapes=[pltpu.VMEM((8, F), jnp.float32)],
  )(sm, mx, ct, xs)


def _norm_apply_tc(x1, stats, weight, bias, mscale, bm=1000):
  F = 2 * _D

  def body(x_ref, st_ref, w_ref, b_ref, ms_ref, o_ref):
    mean = st_ref[0:1, :] / _N
    ex2 = st_ref[1:2, :] / _N
    ctr = ms_ref[...] * mean
    var = ex2 - 2.0 * ctr * mean + ctr * ctr
    scale = w_ref[...] * jax.lax.rsqrt(var + 1e-5)
    shift = b_ref[...] - ctr * scale
    o_ref[...] = x_ref[...] * scale + shift

  return pl.pallas_call(
      body, grid=(_N // bm,),
      in_specs=[pl.BlockSpec((bm, F), lambda i: (i, 0)),
                pl.BlockSpec((8, F), lambda i: (0, 0)),
                pl.BlockSpec((1, F), lambda i: (0, 0)),
                pl.BlockSpec((1, F), lambda i: (0, 0)),
                pl.BlockSpec((1, F), lambda i: (0, 0))],
      out_specs=pl.BlockSpec((bm, F), lambda i: (i, 0)),
      out_shape=jax.ShapeDtypeStruct((_N, F), jnp.float32),
  )(x1, stats, weight.reshape(1, F), bias.reshape(1, F),
    mscale.reshape(1, F))


def _alpha_tc(xls, xrd, eg, att, be=1024):
  W = _H * _D

  def body(a_ref, b_ref, c_ref, att_ref, o_ref):
    m = a_ref[...] + b_ref[...] + c_ref[...]
    m = jnp.where(m >= 0, m, 0.2 * m)
    m3 = m.reshape(be, _H, _D) * att_ref[...][None]
    al = jnp.sum(m3, axis=2)
    o_ref[...] = jnp.concatenate(
        [al, jnp.zeros((be, 128 - _H), jnp.float32)], axis=1)

  spec = pl.BlockSpec((be, W), lambda i: (i, 0))
  return pl.pallas_call(
      body, grid=(_EP // be,),
      in_specs=[spec, spec, spec, pl.BlockSpec((_H, _D), lambda i: (0, 0))],
      out_specs=pl.BlockSpec((be, 128), lambda i: (i, 0)),
      out_shape=jax.ShapeDtypeStruct((_EP, 128), jnp.float32),
  )(xls, xrd, eg, att)


def _aexp_tc(alpha, amax_d, be=2048):
  def body(a_ref, m_ref, o_ref):
    o_ref[...] = jnp.exp(a_ref[...] - m_ref[...])

  spec = pl.BlockSpec((be, 128), lambda i: (i, 0))
  return pl.pallas_call(
      body, grid=(_EP // be,), in_specs=[spec, spec], out_specs=spec,
      out_shape=jax.ShapeDtypeStruct((_EP, 128), jnp.float32))(alpha, amax_d)


def _wdiv_tc(aexp, den_d, be=2048):
  def body(a_ref, d_ref, o_ref):
    o_ref[...] = a_ref[...] / (d_ref[...] + 1e-16)

  spec = pl.BlockSpec((be, 128), lambda i: (i, 0))
  return pl.pallas_call(
      body, grid=(_EP // be,), in_specs=[spec, spec], out_specs=spec,
      out_shape=jax.ShapeDtypeStruct((_EP, 128), jnp.float32))(aexp, den_d)


def _scale_tc(xls, wgt, be=1024):
  W = _H * _D

  def body(x_ref, w_ref, o_ref):
    w8 = w_ref[...][:, :_H]
    o_ref[...] = (x_ref[...].reshape(be, _H, _D)
                  * w8[:, :, None]).reshape(be, W)

  return pl.pallas_call(
      body, grid=(_EP // be,),
      in_specs=[pl.BlockSpec((be, W), lambda i: (i, 0)),
                pl.BlockSpec((be, 128), lambda i: (i, 0))],
      out_specs=pl.BlockSpec((be, W), lambda i: (i, 0)),
      out_shape=jax.ShapeDtypeStruct((_EP, W), jnp.float32))(xls, wgt)


def _pool_tc(g, batch2, Wg, bg, bm=1000):
  W = _H * _D
  nb = _N // bm

  def body(g_ref, b_ref, wg_ref, bg_ref, o_ref, acc, cacc):
    i = pl.program_id(0)

    @pl.when(i == 0)
    def _():
      acc[...] = jnp.zeros_like(acc)
      cacc[...] = jnp.zeros_like(cacc)

    oh = (b_ref[...] == lax.broadcasted_iota(jnp.int32, (1, _G), 1)
          ).astype(jnp.float32)
    acc[...] += lax.dot_general(oh, g_ref[...], (((0,), (0,)), ((), ())),
                                preferred_element_type=jnp.float32)
    cacc[0:1, :] += jnp.sum(oh, axis=0, keepdims=True)

    @pl.when(i == nb - 1)
    def _():
      cnt = jnp.maximum(cacc[0:1, :], 1.0).reshape(_G, 1)
      gm = acc[...] / cnt
      o_ref[...] = jnp.dot(gm, wg_ref[...],
                           preferred_element_type=jnp.float32) + bg_ref[...]

  return pl.pallas_call(
      body, grid=(nb,),
      in_specs=[pl.BlockSpec((bm, W), lambda i: (i, 0)),
                pl.BlockSpec((bm, 1), lambda i: (i, 0)),
                pl.BlockSpec((W, _D), lambda i: (0, 0)),
                pl.BlockSpec((1, _D), lambda i: (0, 0))],
      out_specs=pl.BlockSpec((_G, _D), lambda i: (0, 0)),
      out_shape=jax.ShapeDtypeStruct((_G, _D), jnp.float32),
      scratch_shapes=[pltpu.VMEM((_G, W), jnp.float32),
                      pltpu.VMEM((8, _G), jnp.float32)],
  )(g, batch2, Wg, bg.reshape(1, _D))


def kernel(node_feature, edge_index, edge_feature, batch, params):
  src = edge_index[0].astype(jnp.int32)
  dst = edge_index[1].astype(jnp.int32)
  perm = jnp.argsort(dst)
  dst_r = dst[perm]
  src_r = src[perm]
  ea_r = edge_feature[perm]
  pad = _EP - _E
  dst_s = jnp.concatenate([dst_r, jnp.full((pad,), _NP - 1, jnp.int32)])
  dst_g = jnp.concatenate([dst_r, jnp.zeros((pad,), jnp.int32)])
  src_s = jnp.concatenate([src_r, jnp.zeros((pad,), jnp.int32)])
  ea_s = jnp.concatenate([ea_r, jnp.zeros((pad, _ED), jnp.float32)])
  bounds = (jnp.arange(_NR + 1, dtype=jnp.int32) * _RN).astype(jnp.int32)
  off = jnp.searchsorted(dst_s, bounds).astype(jnp.int32)
  gidx = jnp.clip(jnp.arange(_NR)[:, None] + jnp.arange(16)[None, :],
                  0, _NR)
  off = off[gidx]

  x = node_feature
  for l in range(3):
    p = params['convs'][l]
    k = _mm(x, p['Wk'], p['bk'], 1000)
    q = _mm(x, p['Wq'], p['bq'], 1000)
    v = _mm(x, p['Wv'], p['bv'], 1000)
    xs = _mm(x, p['Ws'], p['bs'], 1000)
    ep = _mm(ea_s, p['We'], p['be'], 2048)
    kd = _sc_gather(k, dst_g, 128)
    qs = _sc_gather(q, src_s, 128)
    vs = _sc_gather(v, src_s, 128)
    msg = _msg_tc(kd, qs, ep, vs)
    sm, mx, ct = _sc_reduce(msg, dst_s, off, 128, True, True)
    x1, stats = _combine_tc(sm[:_N], mx[:_N], ct[:_N], xs, l < 2)
    if l < 2:
      np_ = params['norms'][l]
      x = _norm_apply_tc(x1, stats, np_['weight'], np_['bias'],
                         np_['mean_scale'])
    else:
      x = x1

  nl = params['node_linear']
  local = _mm(x, nl['W'], nl['b'], 1000)

  gp = params['gat']
  xl = _mm(local, gp['Wl'], gp['bl'], 1000)
  xr = _mm(local, gp['Wr'], gp['br'], 1000)
  eg = _mm(ea_s, gp['We'], gp['be'], 2048)
  xls = _sc_gather(xl, src_s, 32)
  xrd = _sc_gather(xr, dst_g, 32)
  alpha = _alpha_tc(xls, xrd, eg, gp['att'])
  (amax,) = _sc_reduce(alpha, dst_s, off, 128, True, False)[1:]
  amax_d = _sc_gather(amax, dst_s, 128)
  aexp = _aexp_tc(alpha, amax_d)
  (den,) = _sc_reduce(aexp, dst_s, off, 128, False, False)
  den_d = _sc_gather(den, dst_s, 128)
  wgt = _wdiv_tc(aexp, den_d)
  smsg = _scale_tc(xls, wgt)
  (g,) = _sc_reduce(smsg, dst_s, off, 32, False, False)

  gl = params['graph_linear']
  glob = _pool_tc(g[:_N], batch.reshape(_N, 1).astype(jnp.int32),
                  gl['W'], gl['b'], 1000)
  return local, glob


# speedup vs baseline: 2.5255x; 1.0281x over previous
import functools

import jax
import jax.numpy as jnp
from jax import lax
from jax.experimental import pallas as pl
from jax.experimental.pallas import tpu as pltpu
from jax.experimental.pallas import tpu_sc as plsc

_N = 10000
_NP = 10240
_E = 160000
_EP = 163840
_D = 128
_ED = 16
_H = 8
_G = 16
_NC = 2
_NS = 16
_NW = _NC * _NS
_RN = 80
_NR = 128
_RPW = _NR // _NW
_AR = _RN + 8

_mesh = plsc.VectorSubcoreMesh(core_axis_name="c", subcore_axis_name="s")


@functools.lru_cache(maxsize=None)
def _gather_fn(V, width, chunk):
  per_w = _EP // _NW
  n_ch = per_w // chunk

  n2 = n_ch // 2

  def body(table_hbm, idx_hbm, out_hbm, idx_v0, idx_v1, rows_v0, rows_v1,
           sem0, sem1):
    wid = lax.axis_index("s") * _NC + lax.axis_index("c")
    base = wid * per_w

    pltpu.sync_copy(idx_hbm.at[pl.ds(base, chunk)], idx_v0)
    pltpu.async_copy(table_hbm.at[idx_v0], rows_v0, sem0)

    def step(ci, c):
      ea = base + (2 * ci) * chunk
      eb = ea + chunk
      pltpu.sync_copy(idx_hbm.at[pl.ds(eb, chunk)], idx_v1)
      pltpu.async_copy(table_hbm.at[idx_v1], rows_v1, sem1)
      pltpu.make_async_copy(table_hbm.at[idx_v0], rows_v0, sem0).wait()
      pltpu.sync_copy(rows_v0, out_hbm.at[pl.ds(ea, chunk)])

      @pl.when(ci < n2 - 1)
      def _():
        ea2 = ea + 2 * chunk
        pltpu.sync_copy(idx_hbm.at[pl.ds(ea2, chunk)], idx_v0)
        pltpu.async_copy(table_hbm.at[idx_v0], rows_v0, sem0)

      pltpu.make_async_copy(table_hbm.at[idx_v1], rows_v1, sem1).wait()
      pltpu.sync_copy(rows_v1, out_hbm.at[pl.ds(eb, chunk)])
      return c

    lax.fori_loop(0, n2, step, 0)

  return pl.kernel(
      body,
      out_type=jax.ShapeDtypeStruct((_EP, width), jnp.float32),
      mesh=_mesh,
      scratch_types=[
          pltpu.VMEM((chunk,), jnp.int32),
          pltpu.VMEM((chunk,), jnp.int32),
          pltpu.VMEM((chunk, width), jnp.float32),
          pltpu.VMEM((chunk, width), jnp.float32),
          pltpu.SemaphoreType.DMA,
          pltpu.SemaphoreType.DMA,
      ])


def _sc_gather(table, idx, chunk):
  return _gather_fn(table.shape[0], table.shape[1], chunk)(table, idx)


@functools.lru_cache(maxsize=None)
def _reduce_fn(width, chunk, do_max, do_cnt, wlim):
  def body(*refs):
    it = iter(refs)
    msg_hbm = next(it); dst_hbm = next(it); off_hbm = next(it)
    zs_hbm = next(it)
    zm_hbm = next(it) if do_max else None
    zc_hbm = next(it) if do_cnt else None
    sum_hbm = next(it)
    max_hbm = next(it) if do_max else None
    cnt_hbm = next(it) if do_cnt else None
    offs_v = next(it); dstv_v = next(it); lrow_v = next(it)
    msg_v = next(it)
    acc_s = next(it)
    acc_m = next(it) if do_max else None
    acc_c = next(it) if do_cnt else None
    sem = next(it)

    wid = lax.axis_index("s") * _NC + lax.axis_index("c")

    def rbody(r, cr):
      ridx = wid * _RPW + r
      base = ridx * _RN
      pltpu.sync_copy(off_hbm.at[ridx], offs_v)
      pltpu.sync_copy(zs_hbm, acc_s)
      if do_max:
        pltpu.sync_copy(zm_hbm, acc_m)
      if do_cnt:
        pltpu.sync_copy(zc_hbm, acc_c)
      orow = offs_v[pl.ds(0, 16)]
      e0 = orow[0]
      e1 = orow[1]
      e0a = (e0 // chunk) * chunk
      nch = (e1 - e0a + chunk - 1) // chunk

      def step(ci, c, e0a=e0a, base=base):
        ea = e0a + ci * chunk
        pltpu.sync_copy(dst_hbm.at[pl.ds(ea, chunk)], dstv_v)
        pltpu.async_copy(msg_hbm.at[pl.ds(ea, chunk)], msg_v, sem).wait()
        for jj in range(chunk // 16):
          dv = dstv_v[pl.ds(jj * 16, 16)]
          lr = dv - base
          ok = jnp.logical_and(lr >= 0, lr < _RN)
          lrow_v[pl.ds(jj * 16, 16)] = jnp.where(ok, lr, _RN)

        def inner(i2, c2):
          rvec = lrow_v[pl.ds(i2 * 16, 16)]
          for kk in range(16):
            row = rvec[kk]
            for j in range(wlim):
              sl = pl.ds(j * 16, 16)
              mv = msg_v[i2 * 16 + kk, sl]
              acc_s[row, sl] += mv
              if do_max:
                acc_m[row, sl] = jnp.maximum(acc_m[row, sl], mv)
            if do_cnt:
              c16 = pl.ds(0, 16)
              acc_c[row, c16] += 1.0
          return c2

        lax.fori_loop(0, chunk // 16, inner, 0)
        return c

      lax.fori_loop(0, nch, step, 0)
      pltpu.sync_copy(acc_s.at[pl.ds(0, _RN)], sum_hbm.at[pl.ds(base, _RN)])
      if do_max:
        pltpu.sync_copy(acc_m.at[pl.ds(0, _RN)], max_hbm.at[pl.ds(base, _RN)])
      if do_cnt:
        pltpu.sync_copy(acc_c.at[pl.ds(0, _RN)], cnt_hbm.at[pl.ds(base, _RN)])
      return cr

    lax.fori_loop(0, _RPW, rbody, 0)

  outs = [jax.ShapeDtypeStruct((_NP, width), jnp.float32)]
  if do_max:
    outs.append(jax.ShapeDtypeStruct((_NP, width), jnp.float32))
  if do_cnt:
    outs.append(jax.ShapeDtypeStruct((_NP, 16), jnp.float32))
  scratch = [
      pltpu.VMEM((16,), jnp.int32),
      pltpu.VMEM((chunk,), jnp.int32),
      pltpu.VMEM((chunk,), jnp.int32),
      pltpu.VMEM((chunk, width), jnp.float32),
      pltpu.VMEM((_AR, width), jnp.float32),
  ]
  if do_max:
    scratch.append(pltpu.VMEM((_AR, width), jnp.float32))
  if do_cnt:
    scratch.append(pltpu.VMEM((_AR, 16), jnp.float32))
  scratch.append(pltpu.SemaphoreType.DMA)

  return pl.kernel(body, out_type=tuple(outs), mesh=_mesh,
                   scratch_types=scratch)


def _sc_reduce(msg, dst_s, off, chunk, do_max, do_cnt, wlim=None):
  width = msg.shape[1]
  if wlim is None:
    wlim = width // 16
  fn = _reduce_fn(width, chunk, do_max, do_cnt, wlim)
  args = [msg, dst_s, off, jnp.zeros((_AR, width), jnp.float32)]
  if do_max:
    args.append(jnp.full((_AR, width), -1e38, jnp.float32))
  if do_cnt:
    args.append(jnp.zeros((_AR, 16), jnp.float32))
  return fn(*args)


def _mm(x, W, b, bm):
  M, K = x.shape
  F = W.shape[1]

  def body(x_ref, w_ref, b_ref, o_ref):
    o_ref[...] = jnp.dot(x_ref[...], w_ref[...],
                         preferred_element_type=jnp.float32) + b_ref[...]

  return pl.pallas_call(
      body,
      grid=(M // bm,),
      in_specs=[pl.BlockSpec((bm, K), lambda i: (i, 0)),
                pl.BlockSpec((K, F), lambda i: (0, 0)),
                pl.BlockSpec((1, F), lambda i: (0, 0))],
      out_specs=pl.BlockSpec((bm, F), lambda i: (i, 0)),
      out_shape=jax.ShapeDtypeStruct((M, F), jnp.float32),
  )(x, W, b.reshape(1, F))


def _msg_tc(kd, qs, ep, vs, be=2048):
  def body(a, b, c, d, o):
    o[...] = jax.nn.sigmoid(a[...] + b[...] + c[...]) * d[...]

  spec = pl.BlockSpec((be, _D), lambda i: (i, 0))
  return pl.pallas_call(
      body, grid=(_EP // be,),
      in_specs=[spec] * 4, out_specs=spec,
      out_shape=jax.ShapeDtypeStruct((_EP, _D), jnp.float32))(kd, qs, ep, vs)


def _combine_tc(sm, mx, ct, xs, do_stats, bm=1000):
  nb = _N // bm
  F = 2 * _D

  def body(s_ref, m_ref, c_ref, xs_ref, o_ref, st_ref, acc):
    i = pl.program_id(0)
    deg = jnp.maximum(c_ref[...][:, :1], 1.0)
    mean = s_ref[...] / deg
    mraw = m_ref[...]
    mv = jnp.where(mraw < -9e37, 0.0, mraw)
    x1 = jnp.concatenate([mean, mv], axis=1) + xs_ref[...]
    if do_stats:
      x1 = x1 * jnp.tanh(jax.nn.softplus(x1))

      @pl.when(i == 0)
      def _():
        acc[...] = jnp.zeros_like(acc)

      acc[0:1, :] += jnp.sum(x1, axis=0, keepdims=True)
      acc[1:2, :] += jnp.sum(x1 * x1, axis=0, keepdims=True)

      @pl.when(i == nb - 1)
      def _():
        st_ref[...] = acc[...]
    else:
      @pl.when(i == 0)
      def _():
        st_ref[...] = jnp.zeros_like(st_ref)
    o_ref[...] = x1

  return pl.pallas_call(
      body, grid=(nb,),
      in_specs=[pl.BlockSpec((bm, _D), lambda i: (i, 0)),
                pl.BlockSpec((bm, _D), lambda i: (i, 0)),
                pl.BlockSpec((bm, 16), lambda i: (i, 0)),
                pl.BlockSpec((bm, F), lambda i: (i, 0))],
      out_specs=[pl.BlockSpec((bm, F), lambda i: (i, 0)),
                 pl.BlockSpec((8, F), lambda i: (0, 0))],
      out_shape=[jax.ShapeDtypeStruct((_N, F), jnp.float32),
                 jax.ShapeDtypeStruct((8, F), jnp.float32)],
      scratch_shapes=[pltpu.VMEM((8, F), jnp.float32)],
  )(sm, mx, ct, xs)


def _norm_apply_tc(x1, stats, weight, bias, mscale, bm=1000):
  F = 2 * _D

  def body(x_ref, st_ref, w_ref, b_ref, ms_ref, o_ref):
    mean = st_ref[0:1, :] / _N
    ex2 = st_ref[1:2, :] / _N
    ctr = ms_ref[...] * mean
    var = ex2 - 2.0 * ctr * mean + ctr * ctr
    scale = w_ref[...] * jax.lax.rsqrt(var + 1e-5)
    shift = b_ref[...] - ctr * scale
    o_ref[...] = x_ref[...] * scale + shift

  return pl.pallas_call(
      body, grid=(_N // bm,),
      in_specs=[pl.BlockSpec((bm, F), lambda i: (i, 0)),
                pl.BlockSpec((8, F), lambda i: (0, 0)),
                pl.BlockSpec((1, F), lambda i: (0, 0)),
                pl.BlockSpec((1, F), lambda i: (0, 0)),
                pl.BlockSpec((1, F), lambda i: (0, 0))],
      out_specs=pl.BlockSpec((bm, F), lambda i: (i, 0)),
      out_shape=jax.ShapeDtypeStruct((_N, F), jnp.float32),
  )(x1, stats, weight.reshape(1, F), bias.reshape(1, F),
    mscale.reshape(1, F))


def _alpha_tc(xls, xrd, eg, att, be=1024):
  W = _H * _D

  def body(a_ref, b_ref, c_ref, att_ref, o_ref):
    m = a_ref[...] + b_ref[...] + c_ref[...]
    m = jnp.where(m >= 0, m, 0.2 * m)
    m3 = m.reshape(be, _H, _D) * att_ref[...][None]
    al = jnp.sum(m3, axis=2)
    o_ref[...] = jnp.concatenate(
        [al, jnp.zeros((be, 128 - _H), jnp.float32)], axis=1)

  spec = pl.BlockSpec((be, W), lambda i: (i, 0))
  return pl.pallas_call(
      body, grid=(_EP // be,),
      in_specs=[spec, spec, spec, pl.BlockSpec((_H, _D), lambda i: (0, 0))],
      out_specs=pl.BlockSpec((be, 128), lambda i: (i, 0)),
      out_shape=jax.ShapeDtypeStruct((_EP, 128), jnp.float32),
  )(xls, xrd, eg, att)


def _aexp_tc(alpha, amax_d, be=2048):
  def body(a_ref, m_ref, o_ref):
    o_ref[...] = jnp.exp(a_ref[...] - m_ref[...])

  spec = pl.BlockSpec((be, 128), lambda i: (i, 0))
  return pl.pallas_call(
      body, grid=(_EP // be,), in_specs=[spec, spec], out_specs=spec,
      out_shape=jax.ShapeDtypeStruct((_EP, 128), jnp.float32))(alpha, amax_d)


def _wdiv_tc(aexp, den_d, be=2048):
  def body(a_ref, d_ref, o_ref):
    o_ref[...] = a_ref[...] / (d_ref[...] + 1e-16)

  spec = pl.BlockSpec((be, 128), lambda i: (i, 0))
  return pl.pallas_call(
      body, grid=(_EP // be,), in_specs=[spec, spec], out_specs=spec,
      out_shape=jax.ShapeDtypeStruct((_EP, 128), jnp.float32))(aexp, den_d)


def _scale_tc(xls, wgt, be=1024):
  W = _H * _D

  def body(x_ref, w_ref, o_ref):
    w8 = w_ref[...][:, :_H]
    o_ref[...] = (x_ref[...].reshape(be, _H, _D)
                  * w8[:, :, None]).reshape(be, W)

  return pl.pallas_call(
      body, grid=(_EP // be,),
      in_specs=[pl.BlockSpec((be, W), lambda i: (i, 0)),
                pl.BlockSpec((be, 128), lambda i: (i, 0))],
      out_specs=pl.BlockSpec((be, W), lambda i: (i, 0)),
      out_shape=jax.ShapeDtypeStruct((_EP, W), jnp.float32))(xls, wgt)


def _pool_tc(g, batch2, Wg, bg, bm=1000):
  W = _H * _D
  nb = _N // bm

  def body(g_ref, b_ref, wg_ref, bg_ref, o_ref, acc, cacc):
    i = pl.program_id(0)

    @pl.when(i == 0)
    def _():
      acc[...] = jnp.zeros_like(acc)
      cacc[...] = jnp.zeros_like(cacc)

    oh = (b_ref[...] == lax.broadcasted_iota(jnp.int32, (1, _G), 1)
          ).astype(jnp.float32)
    acc[...] += lax.dot_general(oh, g_ref[...], (((0,), (0,)), ((), ())),
                                preferred_element_type=jnp.float32)
    cacc[0:1, :] += jnp.sum(oh, axis=0, keepdims=True)

    @pl.when(i == nb - 1)
    def _():
      cnt = jnp.maximum(cacc[0:1, :], 1.0).reshape(_G, 1)
      gm = acc[...] / cnt
      o_ref[...] = jnp.dot(gm, wg_ref[...],
                           preferred_element_type=jnp.float32) + bg_ref[...]

  return pl.pallas_call(
      body, grid=(nb,),
      in_specs=[pl.BlockSpec((bm, W), lambda i: (i, 0)),
                pl.BlockSpec((bm, 1), lambda i: (i, 0)),
                pl.BlockSpec((W, _D), lambda i: (0, 0)),
                pl.BlockSpec((1, _D), lambda i: (0, 0))],
      out_specs=pl.BlockSpec((_G, _D), lambda i: (0, 0)),
      out_shape=jax.ShapeDtypeStruct((_G, _D), jnp.float32),
      scratch_shapes=[pltpu.VMEM((_G, W), jnp.float32),
                      pltpu.VMEM((8, _G), jnp.float32)],
  )(g, batch2, Wg, bg.reshape(1, _D))


def kernel(node_feature, edge_index, edge_feature, batch, params):
  src = edge_index[0].astype(jnp.int32)
  dst = edge_index[1].astype(jnp.int32)
  perm = jnp.argsort(dst)
  dst_r = dst[perm]
  src_r = src[perm]
  ea_r = edge_feature[perm]
  pad = _EP - _E
  dst_s = jnp.concatenate([dst_r, jnp.full((pad,), _NP - 1, jnp.int32)])
  dst_g = jnp.concatenate([dst_r, jnp.zeros((pad,), jnp.int32)])
  src_s = jnp.concatenate([src_r, jnp.zeros((pad,), jnp.int32)])
  ea_s = jnp.concatenate([ea_r, jnp.zeros((pad, _ED), jnp.float32)])
  bounds = (jnp.arange(_NR + 1, dtype=jnp.int32) * _RN).astype(jnp.int32)
  off = jnp.searchsorted(dst_s, bounds).astype(jnp.int32)
  gidx = jnp.clip(jnp.arange(_NR)[:, None] + jnp.arange(16)[None, :],
                  0, _NR)
  off = off[gidx]

  x = node_feature
  for l in range(3):
    p = params['convs'][l]
    k = _mm(x, p['Wk'], p['bk'], 1000)
    q = _mm(x, p['Wq'], p['bq'], 1000)
    v = _mm(x, p['Wv'], p['bv'], 1000)
    xs = _mm(x, p['Ws'], p['bs'], 1000)
    ep = _mm(ea_s, p['We'], p['be'], 2048)
    kd = _sc_gather(k, dst_g, 128)
    qs = _sc_gather(q, src_s, 128)
    vs = _sc_gather(v, src_s, 128)
    msg = _msg_tc(kd, qs, ep, vs)
    sm, mx, ct = _sc_reduce(msg, dst_s, off, 128, True, True)
    x1, stats = _combine_tc(sm[:_N], mx[:_N], ct[:_N], xs, l < 2)
    if l < 2:
      np_ = params['norms'][l]
      x = _norm_apply_tc(x1, stats, np_['weight'], np_['bias'],
                         np_['mean_scale'])
    else:
      x = x1

  nl = params['node_linear']
  local = _mm(x, nl['W'], nl['b'], 1000)

  gp = params['gat']
  xl = _mm(local, gp['Wl'], gp['bl'], 1000)
  xr = _mm(local, gp['Wr'], gp['br'], 1000)
  eg = _mm(ea_s, gp['We'], gp['be'], 2048)
  xls = _sc_gather(xl, src_s, 32)
  xrd = _sc_gather(xr, dst_g, 32)
  alpha = _alpha_tc(xls, xrd, eg, gp['att'])
  (amax,) = _sc_reduce(alpha, dst_s, off, 128, True, False, 1)[1:]
  amax_d = _sc_gather(amax, dst_s, 128)
  aexp = _aexp_tc(alpha, amax_d)
  (den,) = _sc_reduce(aexp, dst_s, off, 128, False, False, 1)
  den_d = _sc_gather(den, dst_s, 128)
  wgt = _wdiv_tc(aexp, den_d)
  smsg = _scale_tc(xls, wgt)
  (g,) = _sc_reduce(smsg, dst_s, off, 32, False, False)

  gl = params['graph_linear']
  glob = _pool_tc(g[:_N], batch.reshape(_N, 1).astype(jnp.int32),
                  gl['W'], gl['b'], 1000)
  return local, glob
